# Initial kernel scaffold; baseline (speedup 1.0000x reference)
import jax
import jax.numpy as jnp
from jax.experimental import pallas as pl


def kernel(y, neighbor_weights, neighbor_ids):
    raise NotImplementedError("write your pallas kernel here")


# speedup vs baseline: 113.9380x; 113.9380x over previous
import functools

import jax
import jax.numpy as jnp
from jax import lax
from jax.experimental import pallas as pl
from jax.experimental.pallas import tpu as pltpu
from jax.experimental.pallas import tpu_sc as plsc

_N = 100000
_K = 64
_L = 16
_NBLK = _N // _L


def _body(nw, blk_per_w, extra, y_hbm, w_hbm, ids_hbm, out_hbm,
          y_v, ids_v, w_v, stage_v):
    nc = nw // _L
    c = lax.axis_index("c")
    s = lax.axis_index("s")
    wid = s * nc + c
    nblk = blk_per_w + jnp.where(wid < extra, 1, 0)
    base_blk = wid * blk_per_w + jnp.minimum(wid, extra)

    pltpu.sync_copy(y_hbm, y_v)

    def msum(i, acc):
        a = acc
        for j in range(10):
            a = a + y_v[pl.ds((i * 10 + j) * _L, _L)]
        return a
    sv = lax.fori_loop(0, _NBLK // 10, msum, jnp.zeros((_L,), jnp.float32))
    m = jnp.sum(sv) * jnp.float32(1.0 / _N)

    iota = lax.iota(jnp.int32, _L)
    zeros = jnp.zeros((_L,), jnp.float32)

    def block_body(b, sumabs):
        row0 = (base_blk + b) * _L
        pltpu.sync_copy(ids_hbm.at[pl.ds(row0, _L), :], ids_v)
        pltpu.sync_copy(w_hbm.at[pl.ds(row0, _L), :], w_v)
        accB = zeros
        accC = zeros
        for k in range(_K):
            kv = jnp.full((_L,), k, jnp.int32)
            idv = plsc.load_gather(ids_v, [iota, kv])
            wv = plsc.load_gather(w_v, [iota, kv])
            gv = plsc.load_gather(y_v, [idv])
            z = gv - m
            t = wv * z
            accB = accB + t
            accC = accC + t * z
        yv = y_v[pl.ds(row0, _L)]
        ivec = (yv - m) * jnp.float32(_K - 1) * accB / accC
        return sumabs + jnp.abs(ivec)

    sumabs = lax.fori_loop(0, nblk, block_body, zeros)
    stage_v[...] = sumabs
    pltpu.sync_copy(stage_v, out_hbm.at[wid])


def kernel(y, neighbor_weights, neighbor_ids):
    ids32 = neighbor_ids.astype(jnp.int32)
    info = plsc.get_sparse_core_info()
    nw = info.num_cores * info.num_subcores
    blk_per_w = _NBLK // nw
    extra = _NBLK - blk_per_w * nw
    mesh = plsc.VectorSubcoreMesh(core_axis_name="c", subcore_axis_name="s")
    partials = pl.kernel(
        functools.partial(_body, nw, blk_per_w, extra),
        mesh=mesh,
        compiler_params=pltpu.CompilerParams(needs_layout_passes=False),
        out_type=jax.ShapeDtypeStruct((nw, _L), jnp.float32),
        scratch_types=[
            pltpu.VMEM((_N,), jnp.float32),
            pltpu.VMEM((_L, _K), jnp.int32),
            pltpu.VMEM((_L, _K), jnp.float32),
            pltpu.VMEM((_L,), jnp.float32),
        ],
    )(y, neighbor_weights, ids32)
    return jnp.sum(partials) / jnp.float32(_N)
